# Initial kernel scaffold; baseline (speedup 1.0000x reference)
import jax
import jax.numpy as jnp
from jax.experimental import pallas as pl


def kernel(encoder_hidden_states, encoder_position_bias, decoder_input_ids, decoder_attention_mask, encoder_attention_mask, embedding_weight):
    raise NotImplementedError("write your pallas kernel here")


# speedup vs baseline: 1.0129x; 1.0129x over previous
import functools

import jax
import jax.numpy as jnp
from jax import lax
from jax.experimental import pallas as pl
from jax.experimental.pallas import tpu as pltpu
from jax.experimental.pallas import tpu_sc as plsc

NUM_HEADS = 16
NEG = float(jnp.finfo(jnp.float32).min)


def _mask_body(dec_mask_ref, enc_mask_ref, dec_out_ref, enc_out_ref):
    i = pl.program_id(0)
    _, _, R, S = dec_out_ref.shape
    row = i * R + lax.broadcasted_iota(jnp.int32, (1, 1, R, S), 2)
    col = lax.broadcasted_iota(jnp.int32, (1, 1, R, S), 3)
    causal = jnp.where(col <= row, 1.0, 0.0)
    m = dec_mask_ref[0, :].astype(jnp.float32)[None, None, None, :]
    dec_out_ref[...] = (1.0 - causal * m) * NEG
    e = enc_mask_ref[0, :].astype(jnp.float32)[None, None, None, :]
    enc_out_ref[...] = (1.0 - e) * NEG


def _make_masks(dec_mask, enc_mask):
    _, s_dec = dec_mask.shape
    _, s_enc = enc_mask.shape
    rows_per_step = 256
    grid = s_dec // rows_per_step
    return pl.pallas_call(
        _mask_body,
        grid=(grid,),
        in_specs=[
            pl.BlockSpec((1, s_dec), lambda i: (0, 0)),
            pl.BlockSpec((1, s_enc), lambda i: (0, 0)),
        ],
        out_specs=[
            pl.BlockSpec((1, 1, rows_per_step, s_dec), lambda i: (0, 0, i, 0)),
            pl.BlockSpec((1, 1, 1, s_enc), lambda i: (0, 0, 0, 0)),
        ],
        out_shape=[
            jax.ShapeDtypeStruct((1, 1, s_dec, s_dec), jnp.float32),
            jax.ShapeDtypeStruct((1, 1, 1, s_enc), jnp.float32),
        ],
    )(dec_mask, enc_mask)


@functools.lru_cache(maxsize=None)
def _make_sc_gather(n_tok, d_model):
    info = plsc.get_sparse_core_info()
    nc, ns = info.num_cores, info.num_subcores
    nw = nc * ns
    bpw = n_tok // nw
    mesh = plsc.VectorSubcoreMesh(core_axis_name="c", subcore_axis_name="s")

    @functools.partial(
        pl.kernel,
        mesh=mesh,
        out_type=jax.ShapeDtypeStruct((n_tok, d_model), jnp.float32),
        scratch_types=[
            pltpu.VMEM((bpw,), jnp.int32),
            pltpu.VMEM((bpw, d_model), jnp.float32),
            pltpu.SemaphoreType.DMA,
        ],
    )
    def gather_k(table_hbm, idx_hbm, out_hbm, idx_v, rows_v, sem):
        wid = lax.axis_index("s") * nc + lax.axis_index("c")
        base = wid * bpw
        pltpu.sync_copy(idx_hbm.at[pl.ds(base, bpw)], idx_v)
        pltpu.async_copy(table_hbm.at[idx_v], rows_v, sem).wait()
        pltpu.sync_copy(rows_v, out_hbm.at[pl.ds(base, bpw)])

    return gather_k


def kernel(encoder_hidden_states, encoder_position_bias, decoder_input_ids,
           decoder_attention_mask, encoder_attention_mask, embedding_weight):
    b, s_dec = decoder_input_ids.shape
    vocab, d_model = embedding_weight.shape
    ids_flat = decoder_input_ids.reshape(-1)

    gather_k = _make_sc_gather(b * s_dec, d_model)
    decoder_hidden_states = gather_k(embedding_weight, ids_flat)
    decoder_hidden_states = decoder_hidden_states.reshape(b, s_dec, d_model)

    dec_ext, enc_ext = _make_masks(decoder_attention_mask, encoder_attention_mask)

    decoder_position_bias = jnp.zeros((b, NUM_HEADS, s_dec, 1), dtype=jnp.float32)

    return (encoder_hidden_states, encoder_position_bias, decoder_hidden_states,
            enc_ext, dec_ext, decoder_position_bias)
